# baseline (speedup 1.0000x reference)
import functools

import jax
import jax.numpy as jnp
from jax import lax
from jax.experimental import pallas as pl
from jax.experimental.pallas import tpu as pltpu
from jax.experimental.pallas import tpu_sc as plsc

_D = 64
_B = 16384
_NC = 2
_NS = 16
_NW = _NC * _NS
_BPW = _B // _NW
_NCH = 4
_CH = _BPW // _NCH
_L = 16


def _sc_body(idx_hbm, table_hbm, w_hbm, out_hbm, idx_v, rows_v, w_v, out_v, sem):
    wid = lax.axis_index("s") * _NC + lax.axis_index("c")
    base = wid * _BPW

    pltpu.sync_copy(idx_hbm.at[pl.ds(wid * _NCH, _NCH)], idx_v)
    pltpu.sync_copy(w_hbm, w_v)

    copies = [
        pltpu.async_copy(
            table_hbm.at[idx_v.at[j]], rows_v.at[pl.ds(j * _CH, _CH)], sem
        )
        for j in range(_NCH)
    ]
    for c in copies:
        c.wait()

    iota = lax.iota(jnp.int32, _L)
    for g in range(_BPW // _L):
        out_v[pl.ds(g * _L, _L)] = jnp.zeros((_L,), jnp.float32)

    wq = [w_v[pl.ds(k * _L, _L)] for k in range(_D // _L)]

    def body(d, carry):
        dvec = jnp.full((_L,), d, jnp.int32)
        s = jnp.float32(0)
        for k in range(_D // _L):
            mask = (iota + k * _L) == dvec
            s = s + jnp.sum(jnp.where(mask, wq[k], jnp.float32(0)))
        wd = jnp.full((_L,), s)
        for g in range(_BPW // _L):
            rows_idx = g * _L + iota
            vals = plsc.load_gather(rows_v, [rows_idx, dvec])
            plsc.addupdate(out_v.at[pl.ds(g * _L, _L)], vals * wd)
        return carry

    lax.fori_loop(0, _D, body, 0)

    pltpu.sync_copy(out_v, out_hbm.at[pl.ds(base, _BPW)])


_gather_reduce = functools.partial(
    pl.kernel,
    mesh=plsc.VectorSubcoreMesh(core_axis_name="c", subcore_axis_name="s"),
    out_type=jax.ShapeDtypeStruct((_B,), jnp.float32),
    compiler_params=pltpu.CompilerParams(
        needs_layout_passes=False, use_tc_tiling_on_sc=False
    ),
    scratch_types=[
        pltpu.VMEM((_NCH, _CH), jnp.int32),
        pltpu.VMEM((_BPW, _D), jnp.float32),
        pltpu.VMEM((2 * _D,), jnp.float32),
        pltpu.VMEM((_BPW,), jnp.float32),
        pltpu.SemaphoreType.DMA,
    ],
)(_sc_body)


@jax.jit
def kernel(indices, table, W):
    idx2 = indices.reshape(_NW * _NCH, _CH)
    w = jnp.pad(W.reshape(_D), (0, _D))
    out = _gather_reduce(idx2, table, w)
    return out.reshape(_B, 1)


# speedup vs baseline: 1.0328x; 1.0328x over previous
import functools

import jax
import jax.numpy as jnp
from jax import lax
from jax.experimental import pallas as pl
from jax.experimental.pallas import tpu as pltpu
from jax.experimental.pallas import tpu_sc as plsc

_D = 64
_B = 16384
_NC = 2
_NS = 16
_NW = _NC * _NS
_BPW = _B // _NW
_NCH = 4
_CH = _BPW // _NCH
_L = 16
_SR = 2 * _D


def _sc_body(idx_hbm, table_hbm, w_hbm, out_hbm, idx_v, sidx_v, rows_v, w_v,
             out_v, sem):
    wid = lax.axis_index("s") * _NC + lax.axis_index("c")
    base = wid * _BPW

    pltpu.sync_copy(idx_hbm.at[pl.ds(base, _BPW)], idx_v)
    pltpu.sync_copy(w_hbm, w_v)

    for j in range(_NCH):
        for k in range(_CH // _L):
            v = idx_v[pl.ds(j * _CH + k * _L, _L)]
            sidx_v[j, pl.ds(k * _L, _L)] = v >> 1

    copies = [
        pltpu.async_copy(
            table_hbm.at[sidx_v.at[j]], rows_v.at[pl.ds(j * _CH, _CH)], sem
        )
        for j in range(_NCH)
    ]
    for c in copies:
        c.wait()

    iota = lax.iota(jnp.int32, _L)
    wqs = [w_v[pl.ds(k * _L, _L)] for k in range(_D // _L)]

    def group(g, carry):
        hvec = idx_v[pl.ds(g * _L, _L)] & 1
        lo_acc = jnp.zeros((_L,), jnp.float32)
        hi_acc = jnp.zeros((_L,), jnp.float32)
        for j in range(_L):
            row = g * _L + j
            lo = jnp.zeros((_L,), jnp.float32)
            hi = jnp.zeros((_L,), jnp.float32)
            for k in range(_D // _L):
                lo = lo + rows_v[row, pl.ds(k * _L, _L)] * wqs[k]
                hi = hi + rows_v[row, pl.ds(_D + k * _L, _L)] * wqs[k]
            slo = jnp.full((_L,), jnp.sum(lo))
            shi = jnp.full((_L,), jnp.sum(hi))
            sel = iota == j
            lo_acc = jnp.where(sel, slo, lo_acc)
            hi_acc = jnp.where(sel, shi, hi_acc)
        out_v[pl.ds(g * _L, _L)] = jnp.where(hvec == 1, hi_acc, lo_acc)
        return carry

    lax.fori_loop(0, _BPW // _L, group, 0)

    pltpu.sync_copy(out_v, out_hbm.at[pl.ds(base, _BPW)])


_gather_reduce = functools.partial(
    pl.kernel,
    mesh=plsc.VectorSubcoreMesh(core_axis_name="c", subcore_axis_name="s"),
    out_type=jax.ShapeDtypeStruct((_B,), jnp.float32),
    compiler_params=pltpu.CompilerParams(needs_layout_passes=False),
    scratch_types=[
        pltpu.VMEM((_BPW,), jnp.int32),
        pltpu.VMEM((_NCH, _CH), jnp.int32),
        pltpu.VMEM((_BPW, _SR), jnp.float32),
        pltpu.VMEM((_SR,), jnp.float32),
        pltpu.VMEM((_BPW,), jnp.float32),
        pltpu.SemaphoreType.DMA,
    ],
)(_sc_body)


@jax.jit
def kernel(indices, table, W):
    table2 = table.reshape(-1, _SR)
    w = jnp.pad(W.reshape(_D), (0, _D))
    out = _gather_reduce(indices, table2, w)
    return out.reshape(_B, 1)


# speedup vs baseline: 1.7281x; 1.6733x over previous
import functools

import jax
import jax.numpy as jnp
from jax import lax
from jax.experimental import pallas as pl
from jax.experimental.pallas import tpu as pltpu
from jax.experimental.pallas import tpu_sc as plsc

_D = 64
_B = 16384
_NC = 2
_NS = 16
_NW = _NC * _NS
_BPW = _B // _NW
_L = 16


def _sc_body(idx_hbm, table_hbm, w_hbm, out_hbm, idx_v, rows_v, w_v,
             out_v, sem):
    wid = lax.axis_index("s") * _NC + lax.axis_index("c")
    base = wid * _BPW

    pltpu.sync_copy(idx_hbm.at[pl.ds(base, _BPW)], idx_v)
    pltpu.sync_copy(w_hbm, w_v)

    def issue(j, carry):
        ivec = plsc.load_gather(idx_v, [jnp.full((_L,), j)])
        i = jnp.max(ivec)
        pltpu.async_copy(table_hbm.at[i], rows_v.at[j], sem)
        return carry

    lax.fori_loop(0, _BPW, issue, 0)
    pltpu.make_async_copy(table_hbm.at[pl.ds(0, _BPW)], rows_v, sem).wait()

    iota = lax.iota(jnp.int32, _L)
    wqs = [w_v[pl.ds(k * _L, _L)] for k in range(_D // _L)]
    tail = iota == (_L - 1)

    def body(j, carry):
        acc = jnp.zeros((_L,), jnp.float32)
        for k in range(_D // _L):
            acc = acc + rows_v[j, pl.ds(k * _L, _L)] * wqs[k]
        tot = plsc.cumsum(acc)
        plsc.store_scatter(out_v, [jnp.full((_L,), j)], tot, mask=tail)
        return carry

    lax.fori_loop(0, _BPW, body, 0)

    pltpu.sync_copy(out_v, out_hbm.at[pl.ds(base, _BPW)])


_gather_reduce = functools.partial(
    pl.kernel,
    mesh=plsc.VectorSubcoreMesh(core_axis_name="c", subcore_axis_name="s"),
    out_type=jax.ShapeDtypeStruct((_B,), jnp.float32),
    compiler_params=pltpu.CompilerParams(needs_layout_passes=False),
    scratch_types=[
        pltpu.VMEM((_BPW,), jnp.int32),
        pltpu.VMEM((_BPW, _D), jnp.float32),
        pltpu.VMEM((2 * _D,), jnp.float32),
        pltpu.VMEM((_BPW,), jnp.float32),
        pltpu.SemaphoreType.DMA,
    ],
)(_sc_body)


@jax.jit
def kernel(indices, table, W):
    w = jnp.pad(W.reshape(_D), (0, _D))
    out = _gather_reduce(indices, table, w)
    return out.reshape(_B, 1)
